# baseline (speedup 1.0000x reference)
import functools

import jax
import jax.numpy as jnp
from jax import lax
from jax.experimental import pallas as pl
from jax.experimental.pallas import tpu as pltpu
from jax.experimental.pallas import tpu_sc as plsc

N = 10000
NPAD = 10240
E = 320000
DIN = 128
DOUT = 64
COLH = DOUT // 2
K_HOPS = 3

NC = 2
NS = 16
CH = 128
CPT = 160
EPAD = NS * CPT * CH
RPT = NPAD // NS
QR = RPT // 8
NQ = RPT // QR

_mesh = plsc.VectorSubcoreMesh(core_axis_name="c", subcore_axis_name="s")
_sc_params = pltpu.CompilerParams(use_tc_tiling_on_sc=False,
                                  needs_layout_passes=False)


def _newton_rsqrt(x):
    i = plsc.bitcast(x, jnp.int32)
    i = jnp.int32(0x5F3759DF) - lax.shift_right_logical(i, 1)
    y = plsc.bitcast(i, jnp.float32)
    for _ in range(3):
        y = y * (1.5 - 0.5 * x * y * y)
    return y


@functools.partial(
    pl.kernel,
    out_type=(
        jax.ShapeDtypeStruct((NC * NPAD, COLH), jnp.float32),
        jax.ShapeDtypeStruct((NC * NPAD, COLH), jnp.float32),
        jax.ShapeDtypeStruct((NC * NPAD, COLH), jnp.float32),
    ),
    mesh=_mesh,
    scratch_types=[
        pltpu.VMEM((CPT, CH), jnp.int32),
        pltpu.VMEM((CPT, CH), jnp.int32),
        pltpu.VMEM((8, CH, COLH), jnp.float32),
        pltpu.VMEM((CH, 16), jnp.float32),
        pltpu.VMEM((RPT, 16), jnp.float32),
        pltpu.VMEM((QR, COLH), jnp.float32),
        pltpu.VMEM((QR, COLH), jnp.float32),
        pltpu.VMEM((QR, COLH), jnp.float32),
        pltpu.VMEM((QR, 16), jnp.float32),
        pltpu.VMEM((8, DOUT), jnp.float32),
        pltpu.VMEM_SHARED((NPAD, COLH), jnp.float32),
        pltpu.VMEM_SHARED((NPAD, 16), jnp.float32),
        pltpu.SemaphoreType.DMA,
        pltpu.SemaphoreType.DMA,
        pltpu.SemaphoreType.DMA,
        pltpu.SemaphoreType.DMA,
        pltpu.SemaphoreType.DMA,
        pltpu.SemaphoreType.DMA,
        pltpu.SemaphoreType.DMA,
        pltpu.SemaphoreType.DMA,
    ],
    compiler_params=_sc_params,
)
def _sc_mega(xw_hbm, srcr_hbm, dstr_hbm, bv_hbm,
             out_hbm, ping_hbm, pong_hbm,
             src_v, dst_v, gbuf, obuf, dbuf, astage, hstage, zbuf, zbuf16,
             bvv, acc_sp, deg_sp,
             sem0, sem1, sem2, sem3, sem4, sem5, sem6, sem7):
    cid = lax.axis_index("c")
    sid = lax.axis_index("s")
    base = sid * RPT
    gofs = cid * NPAD

    pltpu.sync_copy(srcr_hbm.at[sid], src_v)
    pltpu.sync_copy(dstr_hbm.at[sid], dst_v)
    pltpu.sync_copy(bv_hbm, bvv)

    goff_v = jnp.full((16,), gofs, dtype=jnp.int32)

    def add_off(r, _):
        for j in range(CH // 16):
            sl = pl.ds(j * 16, 16)
            src_v[r, sl] = src_v[r, sl] + goff_v
        return 0

    lax.fori_loop(0, CPT, add_off, 0)

    ones16 = jnp.full((16,), 1.0, dtype=jnp.float32)
    zeros16 = jnp.zeros((16,), jnp.float32)

    def fill_obuf(r, _):
        obuf[r, :] = ones16
        return 0

    lax.fori_loop(0, CH, fill_obuf, 0)

    def fill_z(r, _):
        zbuf16[r, :] = zeros16
        for j in range(COLH // 16):
            zbuf[r, pl.ds(j * 16, 16)] = zeros16
        return 0

    lax.fori_loop(0, QR, fill_z, 0)
    for q in range(NQ):
        pltpu.sync_copy(zbuf, acc_sp.at[pl.ds(base + q * QR, QR)])
        pltpu.sync_copy(zbuf16, deg_sp.at[pl.ds(base + q * QR, QR)])
    plsc.subcore_barrier()

    def dchunk(ci, _):
        pltpu.sync_copy(obuf, deg_sp.at[dst_v.at[ci]], add=True)
        return 0

    lax.fori_loop(0, CPT, dchunk, 0)
    plsc.subcore_barrier()
    pltpu.sync_copy(deg_sp.at[pl.ds(base, RPT)], dbuf)

    for q in range(NQ):
        sl = pl.ds(gofs + base + q * QR, QR)
        pltpu.sync_copy(xw_hbm.at[sl], hstage)

        def scale_row(r, _):
            deg = dbuf[q * QR + r, :] + 1.0
            s = _newton_rsqrt(deg)
            for j in range(COLH // 16):
                csl = pl.ds(j * 16, 16)
                hstage[r, csl] = hstage[r, csl] * s
            return 0

        lax.fori_loop(0, QR, scale_row, 0)
        pltpu.sync_copy(hstage, ping_hbm.at[sl])
    plsc.subcore_barrier()

    sems = (sem0, sem1, sem2, sem3, sem4, sem5, sem6, sem7)
    NB = 8
    for hop in range(K_HOPS):
        cur = ping_hbm if hop % 2 == 0 else pong_hbm
        nxt = pong_hbm if hop % 2 == 0 else ping_hbm
        last = hop == K_HOPS - 1

        for b in range(NB):
            pltpu.async_copy(cur.at[src_v.at[b]], gbuf.at[b], sems[b])

        def outer(j, _):
            for b in range(NB):
                ci = j * NB + b
                pltpu.make_async_copy(cur.at[src_v.at[ci]], gbuf.at[b],
                                      sems[b]).wait()
                pltpu.sync_copy(gbuf.at[b], acc_sp.at[dst_v.at[ci]],
                                add=True)

                @pl.when(ci + NB < CPT)
                def _():
                    pltpu.async_copy(cur.at[src_v.at[ci + NB]], gbuf.at[b],
                                     sems[b])
            return 0

        lax.fori_loop(0, CPT // NB, outer, 0)
        plsc.subcore_barrier()

        for q in range(NQ):
            asl = pl.ds(base + q * QR, QR)
            gsl = pl.ds(gofs + base + q * QR, QR)
            pltpu.sync_copy(acc_sp.at[asl], astage)
            pltpu.sync_copy(cur.at[gsl], hstage)

            def comb_row(r, _):
                deg = dbuf[q * QR + r, :] + 1.0
                if last:
                    sc = _newton_rsqrt(deg)
                else:
                    sc = 1.0 / deg
                for j in range(COLH // 16):
                    csl = pl.ds(j * 16, 16)
                    v = (astage[r, csl] + hstage[r, csl]) * sc
                    if last:
                        v = v + bvv[0, pl.ds(cid * COLH + j * 16, 16)]
                    astage[r, csl] = v
                return 0

            lax.fori_loop(0, QR, comb_row, 0)
            dst_ref = out_hbm if last else nxt
            pltpu.sync_copy(astage, dst_ref.at[gsl])
            if not last:
                pltpu.sync_copy(zbuf, acc_sp.at[asl])
        if not last:
            plsc.subcore_barrier()


def _tc_w_body(w1_ref, w2_ref, b1_ref, b2_ref, w_ref, bv_ref):
    w_ref[...] = jnp.dot(w1_ref[...], w2_ref[...],
                         preferred_element_type=jnp.float32)
    bv_ref[...] = jnp.dot(b1_ref[...], w2_ref[...],
                          preferred_element_type=jnp.float32) + b2_ref[...]


_tc_w = pl.pallas_call(
    _tc_w_body,
    out_shape=(
        jax.ShapeDtypeStruct((DIN, DOUT), jnp.float32),
        jax.ShapeDtypeStruct((8, DOUT), jnp.float32),
    ),
)


def _tc_xw2_body(x_ref, w_ref, o_ref):
    o_ref[...] = jnp.dot(x_ref[...], w_ref[0],
                         preferred_element_type=jnp.float32)


_tc_xw2 = pl.pallas_call(
    _tc_xw2_body,
    grid=(NC, NPAD // 128),
    in_specs=[
        pl.BlockSpec((128, DIN), lambda c, i: (i, 0)),
        pl.BlockSpec((1, DIN, COLH), lambda c, i: (c, 0, 0)),
    ],
    out_specs=pl.BlockSpec((128, COLH), lambda c, i: (c * (NPAD // 128) + i, 0)),
    out_shape=jax.ShapeDtypeStruct((NC * NPAD, COLH), jnp.float32),
)


@jax.jit
def kernel(x, edge_index, W1, b1, W2, b2):
    src = edge_index[0].astype(jnp.int32)
    dst = edge_index[1].astype(jnp.int32)
    ar = jnp.arange(EPAD - E, dtype=jnp.int32)
    pad_src = ar % NPAD
    pad_dst = N + (ar % (NPAD - N))
    srcr = jnp.concatenate([src, pad_src]).reshape(NS, CPT, CH)
    dstr = jnp.concatenate([dst, pad_dst]).reshape(NS, CPT, CH)
    xp = jnp.pad(x, ((0, NPAD - N), (0, 0)))
    b1r = jnp.broadcast_to(b1[None, :], (8, DIN))
    b2r = jnp.broadcast_to(b2[None, :], (8, DOUT))

    w_f, bv = _tc_w(W1, W2, b1r, b2r)
    w_s = jnp.stack([w_f[:, :COLH], w_f[:, COLH:]])
    xw2 = _tc_xw2(xp, w_s)
    outf, _, _ = _sc_mega(xw2, srcr, dstr, bv)
    out = jnp.concatenate(
        [outf[:NPAD], outf[NPAD:]], axis=1)
    return out[:N]
